# baseline (speedup 1.0000x reference)
import functools

import jax
import jax.numpy as jnp
from jax import lax
from jax.experimental import pallas as pl
from jax.experimental.pallas import tpu as pltpu
from jax.experimental.pallas import tpu_sc as plsc

NUM_DAGS = 100000
P = 32
BATCH = 16384
NC, NS = 2, 16
BC = 4096
NQ = 4
KSTART = (0, 24960, 49920, 74880)
KLEN = (24960, 24960, 24960, 25120)

_mesh = plsc.VectorSubcoreMesh(core_axis_name="c", subcore_axis_name="s")


@functools.partial(
    pl.kernel,
    mesh=_mesh,
    out_type=jax.ShapeDtypeStruct((P, P, BATCH), jnp.float32),
    compiler_params=pltpu.CompilerParams(needs_layout_passes=False),
    scratch_types=(
        [pltpu.VMEM((KLEN[p],), jnp.float32) for p in range(4)]
        + [pltpu.VMEM((BATCH,), jnp.float32)]
        + [pltpu.SemaphoreType.DMA for _ in range(5)]
    ),
)
def _probe(g_hbm, idx_hbm, out_hbm, p0, p1, p2, p3, stage,
           sm0, sm1, sm2, sm3, osem):
    c_ax = lax.axis_index("c")
    s_ax = lax.axis_index("s")
    i = s_ax * NC + c_ax
    pieces = (p0, p1, p2, p3)
    sems = (sm0, sm1, sm2, sm3)

    def jbody(j, carry):
        for p in range(4):
            pltpu.async_copy(
                g_hbm.at[i, j, pl.ds(KSTART[p], KLEN[p])], pieces[p], sems[p])
        for p in range(4):
            pltpu.make_async_copy(
                g_hbm.at[i, j, pl.ds(KSTART[p], KLEN[p])], pieces[p],
                sems[p]).wait()
        for cq in range(NQ):
            pltpu.sync_copy(
                stage.at[pl.ds(cq * BC, BC)],
                out_hbm.at[i, j, pl.ds(cq * BC, BC)])
        return carry

    lax.fori_loop(0, P, jbody, 0)


def kernel(g, idx):
    g_t = jnp.transpose(g, (1, 2, 0))
    idx32 = idx.astype(jnp.int32)
    del idx32
    out_t = _probe(g_t, idx)
    return jnp.transpose(out_t, (2, 0, 1))


# speedup vs baseline: 1.0576x; 1.0576x over previous
import functools

import jax
import jax.numpy as jnp
from jax import lax
from jax.experimental import pallas as pl
from jax.experimental.pallas import tpu as pltpu
from jax.experimental.pallas import tpu_sc as plsc

NUM_DAGS = 100000
P = 32
BATCH = 16384
NC, NS = 2, 16
NW = NC * NS
KA = 50048
KB = NUM_DAGS - KA
BC = 4096
NQ = BATCH // BC
UNROLL = 8

_mesh = plsc.VectorSubcoreMesh(core_axis_name="c", subcore_axis_name="s")


@functools.partial(
    pl.kernel,
    mesh=_mesh,
    out_type=jax.ShapeDtypeStruct((P, P, BATCH), jnp.float32),
    compiler_params=pltpu.CompilerParams(needs_layout_passes=False),
    scratch_types=(
        pltpu.VMEM_SHARED((BATCH,), jnp.int32),
        pltpu.VMEM((KA,), jnp.float32),
        pltpu.VMEM((KB,), jnp.float32),
        pltpu.VMEM((BC,), jnp.int32),
        pltpu.VMEM((BC,), jnp.int32),
        pltpu.VMEM((BATCH,), jnp.float32),
        pltpu.SemaphoreType.DMA,
        pltpu.SemaphoreType.DMA,
        pltpu.SemaphoreType.DMA,
        pltpu.SemaphoreType.DMA,
        pltpu.SemaphoreType.DMA,
        pltpu.SemaphoreType.DMA,
        pltpu.SemaphoreType.DMA,
        pltpu.SemaphoreType.DMA,
    ),
)
def _row_sweep(g_hbm, idx_hbm, out_hbm, idx_sp, slabA, slabB, idxq0, idxq1,
               stage, ssemA, ssemB, qsem0, qsem1, osem0, osem1, osem2, osem3):
    c_ax = lax.axis_index("c")
    s_ax = lax.axis_index("s")
    i = s_ax * NC + c_ax

    @pl.when(s_ax == 0)
    def _stage_idx():
        pltpu.sync_copy(idx_hbm, idx_sp)

    plsc.subcore_barrier()

    slabs = (slabA, slabB)
    ssems = (ssemA, ssemB)
    idxqs = (idxq0, idxq1)
    qsems = (qsem0, qsem1)
    osems = (osem0, osem1, osem2, osem3)
    klen = (KA, KB)
    iota16 = lax.iota(jnp.int32, 16)

    def slab_slice(j, h):
        return g_hbm.at[i, j, pl.ds(h * KA, klen[h])]

    def q_slice(cq):
        return idx_sp.at[pl.ds(cq * BC, BC)]

    pltpu.async_copy(slab_slice(0, 0), slabA, ssemA)
    pltpu.async_copy(q_slice(0), idxq0, qsem0)

    def jbody(j, carry):
        m = jnp.where(i == j, 0.0, 1.0).astype(jnp.float32)
        for h in (0, 1):
            if h == 0:
                pltpu.async_copy(slab_slice(j, 1), slabB, ssemB)
            else:
                @pl.when(j + 1 < P)
                def _next_row():
                    pltpu.async_copy(slab_slice(j + 1, 0), slabA, ssemA)
            pltpu.make_async_copy(slab_slice(j, h), slabs[h], ssems[h]).wait()

            for cq in range(NQ):
                qb = cq & 1
                pltpu.make_async_copy(q_slice(cq), idxqs[qb], qsems[qb]).wait()
                nq = (cq + 1) % NQ
                pltpu.async_copy(q_slice(nq), idxqs[nq & 1], qsems[nq & 1])

                if h == 0:
                    @pl.when(j > 0)
                    def _drain_out():
                        pltpu.make_async_copy(
                            stage.at[pl.ds(cq * BC, BC)],
                            out_hbm.at[i, j, pl.ds(cq * BC, BC)],
                            osems[cq],
                        ).wait()

                idxq = idxqs[qb]
                base = cq * BC

                @plsc.parallel_loop(0, BC // 16, step=1, unroll=UNROLL)
                def ebody(v):
                    off = v * 16
                    kv = idxq[pl.ds(off, 16)]
                    if h == 0:
                        msk = kv < KA
                        kl = kv
                    else:
                        msk = kv >= KA
                        kl = kv - KA
                    vals = plsc.load_gather(slabs[h], [kl], mask=msk)
                    pos = (base + off) + iota16
                    plsc.store_scatter(stage, [pos], vals * m, mask=msk)

                if h == 1:
                    pltpu.async_copy(
                        stage.at[pl.ds(base, BC)],
                        out_hbm.at[i, j, pl.ds(base, BC)],
                        osems[cq],
                    )
        return carry

    lax.fori_loop(0, P, jbody, 0)
    pltpu.make_async_copy(q_slice(0), idxq0, qsem0).wait()
    for cq in range(NQ):
        pltpu.make_async_copy(
            stage.at[pl.ds(cq * BC, BC)],
            out_hbm.at[i, P - 1, pl.ds(cq * BC, BC)],
            osems[cq],
        ).wait()


def kernel(g, idx):
    g_t = jnp.transpose(g, (1, 2, 0))
    idx32 = idx.astype(jnp.int32)
    out_t = _row_sweep(g_t, idx32)
    return jnp.transpose(out_t, (2, 0, 1))


# speedup vs baseline: 1.0649x; 1.0069x over previous
import functools

import jax
import jax.numpy as jnp
from jax import lax
from jax.experimental import pallas as pl
from jax.experimental.pallas import tpu as pltpu
from jax.experimental.pallas import tpu_sc as plsc

NUM_DAGS = 100000
P = 32
BATCH = 16384
NC, NS = 2, 16
NW = NC * NS
KA = 50048
KB = NUM_DAGS - KA
BC = 4096
NQ = BATCH // BC
UNROLL = 8

_mesh = plsc.VectorSubcoreMesh(core_axis_name="c", subcore_axis_name="s")


@functools.partial(
    pl.kernel,
    mesh=_mesh,
    out_type=jax.ShapeDtypeStruct((P, P, BATCH), jnp.float32),
    compiler_params=pltpu.CompilerParams(needs_layout_passes=False),
    scratch_types=(
        pltpu.VMEM_SHARED((BATCH,), jnp.int32),
        pltpu.VMEM((KA,), jnp.float32),
        pltpu.VMEM((KB,), jnp.float32),
        pltpu.VMEM((BC,), jnp.int32),
        pltpu.VMEM((BC,), jnp.int32),
        pltpu.VMEM((BATCH,), jnp.float32),
        pltpu.SemaphoreType.DMA,
        pltpu.SemaphoreType.DMA,
        pltpu.SemaphoreType.DMA,
        pltpu.SemaphoreType.DMA,
        pltpu.SemaphoreType.DMA,
        pltpu.SemaphoreType.DMA,
        pltpu.SemaphoreType.DMA,
        pltpu.SemaphoreType.DMA,
    ),
)
def _row_sweep(g_hbm, idx_hbm, out_hbm, idx_sp, slabA, slabB, idxq0, idxq1,
               stage, ssemA, ssemB, qsem0, qsem1, osem0, osem1, osem2, osem3):
    c_ax = lax.axis_index("c")
    s_ax = lax.axis_index("s")
    i = s_ax * NC + c_ax

    pltpu.async_copy(g_hbm.at[i, 0, pl.ds(0, KA)], slabA, ssemA)

    @pl.when(s_ax == 0)
    def _stage_idx():
        pltpu.sync_copy(idx_hbm, idx_sp)

    plsc.subcore_barrier()

    slabs = (slabA, slabB)
    ssems = (ssemA, ssemB)
    idxqs = (idxq0, idxq1)
    qsems = (qsem0, qsem1)
    osems = (osem0, osem1, osem2, osem3)
    klen = (KA, KB)
    iota16 = lax.iota(jnp.int32, 16)

    def slab_slice(j, h):
        return g_hbm.at[i, j, pl.ds(h * KA, klen[h])]

    def q_slice(cq):
        return idx_sp.at[pl.ds(cq * BC, BC)]

    pltpu.async_copy(q_slice(0), idxq0, qsem0)

    def jbody(j, carry):
        m = jnp.where(i == j, 0.0, 1.0).astype(jnp.float32)
        for h in (0, 1):
            if h == 0:
                pltpu.async_copy(slab_slice(j, 1), slabB, ssemB)
            else:
                @pl.when(j + 1 < P)
                def _next_row():
                    pltpu.async_copy(slab_slice(j + 1, 0), slabA, ssemA)
            pltpu.make_async_copy(slab_slice(j, h), slabs[h], ssems[h]).wait()

            for cq in range(NQ):
                qb = cq & 1
                pltpu.make_async_copy(q_slice(cq), idxqs[qb], qsems[qb]).wait()
                nq = (cq + 1) % NQ
                pltpu.async_copy(q_slice(nq), idxqs[nq & 1], qsems[nq & 1])

                if h == 0:
                    @pl.when(j > 0)
                    def _drain_out():
                        pltpu.make_async_copy(
                            stage.at[pl.ds(cq * BC, BC)],
                            out_hbm.at[i, j, pl.ds(cq * BC, BC)],
                            osems[cq],
                        ).wait()

                idxq = idxqs[qb]
                base = cq * BC

                @plsc.parallel_loop(0, BC // 16, step=1, unroll=UNROLL)
                def ebody(v):
                    off = v * 16
                    kv = idxq[pl.ds(off, 16)]
                    if h == 0:
                        msk = kv < KA
                        kl = kv
                    else:
                        msk = kv >= KA
                        kl = kv - KA
                    vals = plsc.load_gather(slabs[h], [kl], mask=msk)
                    pos = (base + off) + iota16
                    plsc.store_scatter(stage, [pos], vals * m, mask=msk)

                if h == 1:
                    pltpu.async_copy(
                        stage.at[pl.ds(base, BC)],
                        out_hbm.at[i, j, pl.ds(base, BC)],
                        osems[cq],
                    )
        return carry

    lax.fori_loop(0, P, jbody, 0)
    pltpu.make_async_copy(q_slice(0), idxq0, qsem0).wait()
    for cq in range(NQ):
        pltpu.make_async_copy(
            stage.at[pl.ds(cq * BC, BC)],
            out_hbm.at[i, P - 1, pl.ds(cq * BC, BC)],
            osems[cq],
        ).wait()


def kernel(g, idx):
    g_t = jnp.transpose(g, (1, 2, 0))
    idx32 = idx.astype(jnp.int32)
    out_t = _row_sweep(g_t, idx32)
    return jnp.transpose(out_t, (2, 0, 1))
